# Initial kernel scaffold; baseline (speedup 1.0000x reference)
import jax
import jax.numpy as jnp
from jax import lax
from jax.experimental import pallas as pl
from jax.experimental.pallas import tpu as pltpu
from jax.experimental.pallas import tpu_sc as plsc
import functools

IDX_W = 128
K = 8
MACRO = K * IDX_W


def _make_lookup(n_macros_total, n_workers, D):
    mesh = plsc.VectorSubcoreMesh(core_axis_name="c", subcore_axis_name="s")
    nc = mesh.num_cores
    per_worker = n_macros_total // n_workers

    @functools.partial(
        pl.kernel,
        out_type=jax.ShapeDtypeStruct((n_macros_total, MACRO, D), jnp.float32),
        mesh=mesh,
        scratch_types=[
            pltpu.VMEM((K, IDX_W), jnp.int32),
            pltpu.VMEM((MACRO, D), jnp.float32),
            pltpu.SemaphoreType.DMA,
        ],
    )
    def lookup(idx_hbm, table_hbm, out_hbm, idx_v, rows_v, sem):
        wid = lax.axis_index("s") * nc + lax.axis_index("c")
        base = wid * per_worker

        @pl.loop(0, per_worker)
        def _(m):
            g = base + m
            pltpu.sync_copy(idx_hbm.at[g], idx_v)
            copies = [
                pltpu.async_copy(
                    table_hbm.at[idx_v.at[j]],
                    rows_v.at[pl.ds(j * IDX_W, IDX_W)],
                    sem,
                )
                for j in range(K)
            ]
            for c in copies:
                c.wait()
            pltpu.sync_copy(rows_v, out_hbm.at[g])

    return lookup


def kernel(token_ids, weight):
    B, H = token_ids.shape
    V, D = weight.shape
    total = B * H
    n_workers = 32
    assert total % (n_workers * MACRO) == 0
    n_macros = total // MACRO
    idx = token_ids.astype(jnp.int32).reshape(n_macros, K, IDX_W)
    out = _make_lookup(n_macros, n_workers, D)(idx, weight)
    return out.reshape(B, H, D)


# speedup vs baseline: 1.2583x; 1.2583x over previous
import jax
import jax.numpy as jnp
from jax import lax
from jax.experimental import pallas as pl
from jax.experimental.pallas import tpu as pltpu
from jax.experimental.pallas import tpu_sc as plsc
import functools

IDX_W = 128
K = 8
MACRO = K * IDX_W


def _make_lookup(n_macros_total, n_workers, D):
    mesh = plsc.VectorSubcoreMesh(core_axis_name="c", subcore_axis_name="s")
    nc = mesh.num_cores
    per_worker = n_macros_total // n_workers

    @functools.partial(
        pl.kernel,
        out_type=jax.ShapeDtypeStruct((n_macros_total, MACRO, D), jnp.float32),
        mesh=mesh,
        scratch_types=[
            pltpu.VMEM((K, IDX_W), jnp.int32),
            pltpu.VMEM((MACRO, D), jnp.float32),
            pltpu.SemaphoreType.DMA,
        ],
        compiler_params=pltpu.CompilerParams(use_tc_tiling_on_sc=False),
    )
    def lookup(idx_hbm, table_hbm, out_hbm, idx_v, rows_v, sem):
        wid = lax.axis_index("s") * nc + lax.axis_index("c")
        base = wid * per_worker

        @pl.loop(0, per_worker)
        def _(m):
            g = base + m
            pltpu.sync_copy(idx_hbm.at[g], idx_v)
            copies = [
                pltpu.async_copy(
                    table_hbm.at[idx_v.at[j]],
                    rows_v.at[pl.ds(j * IDX_W, IDX_W)],
                    sem,
                )
                for j in range(K)
            ]
            for c in copies:
                c.wait()
            pltpu.sync_copy(rows_v, out_hbm.at[g])

    return lookup


def kernel(token_ids, weight):
    B, H = token_ids.shape
    V, D = weight.shape
    total = B * H
    n_workers = 32
    assert total % (n_workers * MACRO) == 0
    n_macros = total // MACRO
    idx = token_ids.astype(jnp.int32).reshape(n_macros, K, IDX_W)
    out = _make_lookup(n_macros, n_workers, D)(idx, weight)
    return out.reshape(B, H, D)


# speedup vs baseline: 1.2880x; 1.0236x over previous
import jax
import jax.numpy as jnp
from jax import lax
from jax.experimental import pallas as pl
from jax.experimental.pallas import tpu as pltpu
from jax.experimental.pallas import tpu_sc as plsc
import functools

IDX_W = 128
K = 10
MACRO = K * IDX_W


def _make_lookup(n_macros_total, n_workers, D):
    mesh = plsc.VectorSubcoreMesh(core_axis_name="c", subcore_axis_name="s")
    nc = mesh.num_cores
    per_worker = n_macros_total // n_workers
    assert per_worker % 2 == 0

    @functools.partial(
        pl.kernel,
        out_type=jax.ShapeDtypeStruct((n_macros_total, MACRO, D), jnp.float32),
        mesh=mesh,
        scratch_types=[
            pltpu.VMEM((K, IDX_W), jnp.int32),
            pltpu.VMEM((K, IDX_W), jnp.int32),
            pltpu.VMEM((MACRO, D), jnp.float32),
            pltpu.VMEM((MACRO, D), jnp.float32),
            pltpu.SemaphoreType.DMA,
            pltpu.SemaphoreType.DMA,
            pltpu.SemaphoreType.DMA,
            pltpu.SemaphoreType.DMA,
        ],
        compiler_params=pltpu.CompilerParams(use_tc_tiling_on_sc=False),
    )
    def lookup(idx_hbm, table_hbm, out_hbm, idx0, idx1, rows0, rows1,
               gsem0, gsem1, wsem0, wsem1):
        wid = lax.axis_index("s") * nc + lax.axis_index("c")
        base = wid * per_worker

        def fire_gathers(idx_v, rows_v, sem):
            return [
                pltpu.async_copy(
                    table_hbm.at[idx_v.at[j]],
                    rows_v.at[pl.ds(j * IDX_W, IDX_W)],
                    sem,
                )
                for j in range(K)
            ]

        @pl.loop(0, per_worker, step=2)
        def _(g):
            m0 = base + g
            m1 = m0 + 1
            pltpu.sync_copy(idx_hbm.at[m0], idx0)
            @pl.when(g > 0)
            def _():
                pltpu.make_async_copy(rows0, out_hbm.at[m0], wsem0).wait()
            c0 = fire_gathers(idx0, rows0, gsem0)
            pltpu.sync_copy(idx_hbm.at[m1], idx1)
            @pl.when(g > 0)
            def _():
                pltpu.make_async_copy(rows1, out_hbm.at[m1], wsem1).wait()
            c1 = fire_gathers(idx1, rows1, gsem1)
            for c in c0:
                c.wait()
            pltpu.async_copy(rows0, out_hbm.at[m0], wsem0)
            for c in c1:
                c.wait()
            pltpu.async_copy(rows1, out_hbm.at[m1], wsem1)

        last = base + per_worker - 2
        pltpu.make_async_copy(rows0, out_hbm.at[last], wsem0).wait()
        pltpu.make_async_copy(rows1, out_hbm.at[last + 1], wsem1).wait()

    return lookup


def kernel(token_ids, weight):
    B, H = token_ids.shape
    V, D = weight.shape
    total = B * H
    n_workers = 32
    assert total % (n_workers * MACRO * 2) == 0
    n_macros = total // MACRO
    idx = token_ids.astype(jnp.int32).reshape(n_macros, K, IDX_W)
    out = _make_lookup(n_macros, n_workers, D)(idx, weight)
    return out.reshape(B, H, D)
